# baseline (speedup 1.0000x reference)
import jax
import jax.numpy as jnp
from jax import lax
from jax.experimental import pallas as pl
from jax.experimental.pallas import tpu as pltpu
from jax.experimental.pallas import tpu_sc as plsc

N = 10000
E = 320000
DX = 128
DE = 16
DH = 64
SW = 128
TR = 64

NC = 2
NS = 16
NW = NC * NS
EPT = E // NW
CW = 80
CH = EPT // CW
NPAD = 10240
RPS = NPAD // NS

_mesh = plsc.VectorSubcoreMesh(
    core_axis_name="c", subcore_axis_name="s", num_cores=NC, num_subcores=NS
)

_sc_params = pltpu.CompilerParams(use_tc_tiling_on_sc=False)

_f32 = jnp.float32


def _zero_fill(ref, rows, ncol):
    zv = jnp.zeros((16,), _f32)

    @pl.loop(0, rows, unroll=4)
    def _(r):
        for cc in range(ncol):
            ref[r, pl.ds(cc * 16, 16)] = zv


def _seg_gather_body(xm, srcr, dstr, sx_o, dg_o,
                     src_v, dst_v, gbA, gbB, ones_v, zx, zrow,
                     sx_s, dg_s, gA, gB, sA, sB):
    with_deg = dg_o is not None
    c = lax.axis_index("c")
    s = lax.axis_index("s")
    w = c * NS + s

    ebase = w * EPT
    pltpu.sync_copy(srcr.at[pl.ds(ebase, EPT)], src_v)
    pltpu.sync_copy(dstr.at[pl.ds(ebase, EPT)], dst_v)

    _zero_fill(zx, 160, DH // 16)
    if with_deg:
        _zero_fill(zrow, RPS, DE // 16)
        ov = jnp.ones((16,), _f32)

        @pl.loop(0, CW, unroll=8)
        def _(r):
            ones_v[r, :] = ov

    base = s * RPS
    for k in range(RPS // 160):
        pltpu.sync_copy(zx, sx_s.at[pl.ds(base + k * 160, 160)])
    if with_deg:
        pltpu.sync_copy(zrow, dg_s.at[pl.ds(base, RPS)])
    plsc.subcore_barrier()

    def start_g(j, xb, sem):
        pltpu.async_copy(xm.at[src_v.at[pl.ds(j * CW, CW)]], xb, sem)

    def wait_g(xb, sem):
        pltpu.make_async_copy(xm.at[src_v.at[pl.ds(0, CW)]], xb, sem).wait()

    def start_s(j, xb, sem):
        idx = dst_v.at[pl.ds(j * CW, CW)]
        pltpu.async_copy(xb, sx_s.at[idx], sem, add=True)
        if with_deg:
            pltpu.async_copy(ones_v, dg_s.at[idx], sem, add=True)

    def wait_s(xb, sem):
        pltpu.make_async_copy(xb, sx_s.at[dst_v.at[pl.ds(0, CW)]], sem).wait()
        if with_deg:
            pltpu.make_async_copy(
                ones_v, dg_s.at[dst_v.at[pl.ds(0, CW)]], sem).wait()

    start_g(0, gbA, gA)
    start_g(1, gbB, gB)

    @pl.loop(0, (CH - 3) // 2)
    def _(t):
        j0 = 2 * t
        wait_g(gbA, gA)
        start_s(j0, gbA, sA)
        wait_g(gbB, gB)
        start_s(j0 + 1, gbB, sB)
        wait_s(gbA, sA)
        start_g(j0 + 2, gbA, gA)
        wait_s(gbB, sB)
        start_g(j0 + 3, gbB, gB)

    wait_g(gbA, gA)
    start_s(CH - 3, gbA, sA)
    wait_s(gbA, sA)
    start_g(CH - 1, gbA, gA)
    wait_g(gbB, gB)
    start_s(CH - 2, gbB, sB)
    wait_g(gbA, gA)
    start_s(CH - 1, gbA, sA)
    wait_s(gbA, sA)
    wait_s(gbB, sB)

    plsc.subcore_barrier()

    pltpu.sync_copy(sx_s.at[pl.ds(base, RPS)], sx_o.at[c, pl.ds(base, RPS)])
    if with_deg:
        pltpu.sync_copy(dg_s.at[pl.ds(base, RPS)], dg_o.at[c, pl.ds(base, RPS)])


def _seg_body_xd(xm, srcr, dstr, sx_o, dg_o,
                 src_v, dst_v, gbA, gbB, ones_v, zx, zrow,
                 sx_s, dg_s, gA, gB, sA, sB):
    _seg_gather_body(xm, srcr, dstr, sx_o, dg_o,
                     src_v, dst_v, gbA, gbB, ones_v, zx, zrow,
                     sx_s, dg_s, gA, gB, sA, sB)


def _seg_body_h(hm, srcr, dstr, sh_o,
                src_v, dst_v, gbA, gbB, zx, sh_s, gA, gB, sA, sB):
    _seg_gather_body(hm, srcr, dstr, sh_o, None,
                     src_v, dst_v, gbA, gbB, None, zx, None,
                     sh_s, None, gA, gB, sA, sB)


def _seg_body_se(dstr, ea_hbm, se_o,
                 dst_v, eaA, eaB, zrow, se_s, gA, gB, sA, sB):
    c = lax.axis_index("c")
    s = lax.axis_index("s")
    w = c * NS + s
    ebase = w * EPT
    pltpu.sync_copy(dstr.at[pl.ds(ebase, EPT)], dst_v)

    _zero_fill(zrow, RPS, DE // 16)
    base = s * RPS
    pltpu.sync_copy(zrow, se_s.at[pl.ds(base, RPS)])
    plsc.subcore_barrier()

    def start_g(j, eb, sem):
        off = pl.multiple_of(ebase + j * CW, 16)
        pltpu.async_copy(ea_hbm.at[pl.ds(off, CW)], eb, sem)

    def wait_g(eb, sem):
        pltpu.make_async_copy(ea_hbm.at[pl.ds(0, CW)], eb, sem).wait()

    def start_s(j, eb, sem):
        pltpu.async_copy(eb, se_s.at[dst_v.at[pl.ds(j * CW, CW)]], sem,
                         add=True)

    def wait_s(eb, sem):
        pltpu.make_async_copy(eb, se_s.at[dst_v.at[pl.ds(0, CW)]], sem).wait()

    start_g(0, eaA, gA)
    start_g(1, eaB, gB)

    @pl.loop(0, (CH - 3) // 2)
    def _(t):
        j0 = 2 * t
        wait_g(eaA, gA)
        start_s(j0, eaA, sA)
        wait_g(eaB, gB)
        start_s(j0 + 1, eaB, sB)
        wait_s(eaA, sA)
        start_g(j0 + 2, eaA, gA)
        wait_s(eaB, sB)
        start_g(j0 + 3, eaB, gB)

    wait_g(eaA, gA)
    start_s(CH - 3, eaA, sA)
    wait_s(eaA, sA)
    start_g(CH - 1, eaA, gA)
    wait_g(eaB, gB)
    start_s(CH - 2, eaB, sB)
    wait_g(eaA, gA)
    start_s(CH - 1, eaA, sA)
    wait_s(eaA, sA)
    wait_s(eaB, sB)

    plsc.subcore_barrier()
    pltpu.sync_copy(se_s.at[pl.ds(base, RPS)], se_o.at[c, pl.ds(base, RPS)])


def _seg1x(xm, srcr, dstr):
    return pl.kernel(
        _seg_body_xd,
        out_type=[
            jax.ShapeDtypeStruct((NC, NPAD, DH), _f32),
            jax.ShapeDtypeStruct((NC, NPAD, DE), _f32),
        ],
        mesh=_mesh,
        scratch_types=[
            pltpu.VMEM((EPT,), jnp.int32),
            pltpu.VMEM((EPT,), jnp.int32),
            pltpu.VMEM((CW, DH), _f32),
            pltpu.VMEM((CW, DH), _f32),
            pltpu.VMEM((CW, DE), _f32),
            pltpu.VMEM((160, DH), _f32),
            pltpu.VMEM((RPS, DE), _f32),
            pltpu.VMEM_SHARED((NPAD, DH), _f32),
            pltpu.VMEM_SHARED((NPAD, DE), _f32),
            pltpu.SemaphoreType.DMA,
            pltpu.SemaphoreType.DMA,
            pltpu.SemaphoreType.DMA,
            pltpu.SemaphoreType.DMA,
        ],
        compiler_params=_sc_params,
    )(xm, srcr, dstr)


def _seg1e(dstr, ea):
    return pl.kernel(
        _seg_body_se,
        out_type=[jax.ShapeDtypeStruct((NC, NPAD, DE), _f32)],
        mesh=_mesh,
        scratch_types=[
            pltpu.VMEM((EPT,), jnp.int32),
            pltpu.VMEM((CW, DE), _f32),
            pltpu.VMEM((CW, DE), _f32),
            pltpu.VMEM((RPS, DE), _f32),
            pltpu.VMEM_SHARED((NPAD, DE), _f32),
            pltpu.SemaphoreType.DMA,
            pltpu.SemaphoreType.DMA,
            pltpu.SemaphoreType.DMA,
            pltpu.SemaphoreType.DMA,
        ],
        compiler_params=_sc_params,
    )(dstr, ea)


def _seg2(hm, srcr, dstr):
    return pl.kernel(
        _seg_body_h,
        out_type=[jax.ShapeDtypeStruct((NC, NPAD, DH), _f32)],
        mesh=_mesh,
        scratch_types=[
            pltpu.VMEM((EPT,), jnp.int32),
            pltpu.VMEM((EPT,), jnp.int32),
            pltpu.VMEM((CW, DH), _f32),
            pltpu.VMEM((CW, DH), _f32),
            pltpu.VMEM((160, DH), _f32),
            pltpu.VMEM_SHARED((NPAD, DH), _f32),
            pltpu.SemaphoreType.DMA,
            pltpu.SemaphoreType.DMA,
            pltpu.SemaphoreType.DMA,
            pltpu.SemaphoreType.DMA,
        ],
        compiler_params=_sc_params,
    )(hm, srcr, dstr)


_R = 2000
_G = N // _R


def _dot(a, b):
    return jnp.dot(a, b, preferred_element_type=_f32)


def _tc0_body(x_r, wm_r, xm_r):
    xm_r[...] = _dot(x_r[...], wm_r[...][:DX])


def _tc0(x, Wmsg):
    return pl.pallas_call(
        _tc0_body,
        grid=(_G,),
        in_specs=[
            pl.BlockSpec((_R, DX), lambda i: (i, 0)),
            pl.BlockSpec((DX + DE, DH), lambda i: (0, 0)),
        ],
        out_specs=pl.BlockSpec((_R, DH), lambda i: (i, 0)),
        out_shape=jax.ShapeDtypeStruct((N, DH), _f32),
    )(x, Wmsg)


def _tc1a_body(x_r, sxm_r, dg_r, ws_r, bs_r, bm_r, t1_r):
    deg = dg_r[0, :, 0:1] + dg_r[1, :, 0:1]
    inv = 1.0 / jnp.maximum(deg, 1.0)
    sxm = sxm_r[0] + sxm_r[1]
    t1_r[...] = (_dot(x_r[...], ws_r[...]) + bs_r[...]
                 + (sxm + deg * bm_r[...]) * inv)


def _tc1a(x, sxm, dg, Wself, bself, bmsg):
    return pl.pallas_call(
        _tc1a_body,
        grid=(_G,),
        in_specs=[
            pl.BlockSpec((_R, DX), lambda i: (i, 0)),
            pl.BlockSpec((NC, _R, DH), lambda i: (0, i, 0)),
            pl.BlockSpec((NC, _R, DE), lambda i: (0, i, 0)),
            pl.BlockSpec((DX, DH), lambda i: (0, 0)),
            pl.BlockSpec((1, DH), lambda i: (0, 0)),
            pl.BlockSpec((1, DH), lambda i: (0, 0)),
        ],
        out_specs=pl.BlockSpec((_R, DH), lambda i: (i, 0)),
        out_shape=jax.ShapeDtypeStruct((N, DH), _f32),
    )(x, sxm, dg, Wself, bself, bmsg)


def _tc1b_body(t1_r, se_r, dg_r, wm_r, wm2_r, h1_r, hm_r):
    deg = dg_r[0, :, 0:1] + dg_r[1, :, 0:1]
    inv = 1.0 / jnp.maximum(deg, 1.0)
    se = se_r[0] + se_r[1]
    h = jnp.maximum(t1_r[...] + _dot(se, wm_r[...][DX:]) * inv, 0.0)
    h1_r[...] = h
    hm_r[...] = _dot(h, wm2_r[...][:DH])


def _tc1b(t1, se, dg, Wmsg, Wmsg2):
    return pl.pallas_call(
        _tc1b_body,
        grid=(_G,),
        in_specs=[
            pl.BlockSpec((_R, DH), lambda i: (i, 0)),
            pl.BlockSpec((NC, _R, DE), lambda i: (0, i, 0)),
            pl.BlockSpec((NC, _R, DE), lambda i: (0, i, 0)),
            pl.BlockSpec((DX + DE, DH), lambda i: (0, 0)),
            pl.BlockSpec((DH + DE, DH), lambda i: (0, 0)),
        ],
        out_specs=(
            pl.BlockSpec((_R, DH), lambda i: (i, 0)),
            pl.BlockSpec((_R, DH), lambda i: (i, 0)),
        ),
        out_shape=(
            jax.ShapeDtypeStruct((N, DH), _f32),
            jax.ShapeDtypeStruct((N, DH), _f32),
        ),
    )(t1, se, dg, Wmsg, Wmsg2)


def _tc2_body(h1_r, shm_r, se_r, dg_r, wm_r, bm_r, ws_r, bs_r,
              mw1, mb1, mw2, mb2, sw1, sb1, sw2, sb2,
              tw1, tb1, tw2, tb2, sw_o, tr_o, acc):
    i = pl.program_id(0)

    @pl.when(i == 0)
    def _():
        acc[...] = jnp.zeros_like(acc)

    deg = dg_r[0, :, 0:1] + dg_r[1, :, 0:1]
    inv = 1.0 / jnp.maximum(deg, 1.0)
    se = se_r[0] + se_r[1]
    shm = shm_r[0] + shm_r[1]
    agg = (shm + _dot(se, wm_r[...][DH:]) + deg * bm_r[...]) * inv
    h2 = jnp.maximum(_dot(h1_r[...], ws_r[...]) + bs_r[...] + agg, 0.0)
    acc[...] += jnp.sum(h2, axis=0, keepdims=True)

    @pl.when(i == _G - 1)
    def _():
        g = acc[...] * (1.0 / N)
        z = jnp.maximum(_dot(g, mw1[...]) + mb1[...], 0.0)
        z = jnp.maximum(_dot(z, mw2[...]) + mb2[...], 0.0)
        zs = jnp.maximum(_dot(z, sw1[...]) + sb1[...], 0.0)
        sw_o[...] = jax.nn.sigmoid(_dot(zs, sw2[...]) + sb2[...])
        zt = jnp.maximum(_dot(z, tw1[...]) + tb1[...], 0.0)
        tr_o[...] = jax.nn.sigmoid(_dot(zt, tw2[...]) + tb2[...])


def _tc2(h1, shm, se, dg, Wmsg, bmsg, Wself, bself,
         mW1, mb1, mW2, mb2, sW1, sb1, sW2, sb2, tW1, tb1, tW2, tb2):
    full = lambda a, b: pl.BlockSpec((a, b), lambda i: (0, 0))
    return pl.pallas_call(
        _tc2_body,
        grid=(_G,),
        in_specs=[
            pl.BlockSpec((_R, DH), lambda i: (i, 0)),
            pl.BlockSpec((NC, _R, DH), lambda i: (0, i, 0)),
            pl.BlockSpec((NC, _R, DE), lambda i: (0, i, 0)),
            pl.BlockSpec((NC, _R, DE), lambda i: (0, i, 0)),
            full(DH + DE, DH), full(1, DH), full(DH, DH), full(1, DH),
            full(DH, 256), full(1, 256), full(256, 128), full(1, 128),
            full(128, 64), full(1, 64), full(64, SW), full(1, SW),
            full(128, 64), full(1, 64), full(64, TR), full(1, TR),
        ],
        out_specs=(
            pl.BlockSpec((1, SW), lambda i: (0, 0)),
            pl.BlockSpec((1, TR), lambda i: (0, 0)),
        ),
        out_shape=(
            jax.ShapeDtypeStruct((1, SW), _f32),
            jax.ShapeDtypeStruct((1, TR), _f32),
        ),
        scratch_shapes=[pltpu.VMEM((1, DH), _f32)],
    )(h1, shm, se, dg, Wmsg, bmsg, Wself, bself,
      mW1, mb1, mW2, mb2, sW1, sb1, sW2, sb2, tW1, tb1, tW2, tb2)


def kernel(x, edge_index, edge_attr, g1_Wmsg, g1_bmsg, g1_Wself, g1_bself,
           g2_Wmsg, g2_bmsg, g2_Wself, g2_bself, mlp_W1, mlp_b1, mlp_W2,
           mlp_b2, sw_W1, sw_b1, sw_W2, sw_b2, tr_W1, tr_b1, tr_W2, tr_b2):
    srcr = edge_index[0]
    dstr = edge_index[1]
    row = lambda b: b.reshape(1, -1)

    xm = _tc0(x, g1_Wmsg)
    sxm, dg = _seg1x(xm, srcr, dstr)
    (se,) = _seg1e(dstr, edge_attr)
    t1 = _tc1a(x, sxm, dg, g1_Wself, row(g1_bself), row(g1_bmsg))
    h1, hm = _tc1b(t1, se, dg, g1_Wmsg, g2_Wmsg)
    (shm,) = _seg2(hm, srcr, dstr)
    sw, tr = _tc2(h1, shm, se, dg, g2_Wmsg, row(g2_bmsg), g2_Wself,
                  row(g2_bself), mlp_W1, row(mlp_b1), mlp_W2, row(mlp_b2),
                  sw_W1, row(sw_b1), sw_W2, row(sw_b2),
                  tr_W1, row(tr_b1), tr_W2, row(tr_b2))
    return (sw.reshape(-1), tr.reshape(-1))


# speedup vs baseline: 1.0023x; 1.0023x over previous
import jax
import jax.numpy as jnp
from jax import lax
from jax.experimental import pallas as pl
from jax.experimental.pallas import tpu as pltpu
from jax.experimental.pallas import tpu_sc as plsc

N = 10000
E = 320000
DX = 128
DE = 16
DH = 64
SW = 128
TR = 64

NC = 2
NS = 16
NW = NC * NS
EPT = E // NW
CW = 80
CH = EPT // CW
NPAD = 10240
RPS = NPAD // NS

_mesh = plsc.VectorSubcoreMesh(
    core_axis_name="c", subcore_axis_name="s", num_cores=NC, num_subcores=NS
)

_sc_params = pltpu.CompilerParams(use_tc_tiling_on_sc=False)

_f32 = jnp.float32


def _zero_fill(ref, rows, ncol):
    zv = jnp.zeros((16,), _f32)

    @pl.loop(0, rows, unroll=4)
    def _(r):
        for cc in range(ncol):
            ref[r, pl.ds(cc * 16, 16)] = zv


def _seg_gather_body(xm, srcr, dstr, sx_o, dg_o,
                     src_v, dst_v, gbA, gbB, ones_v, zx, zrow,
                     sx_s, dg_s, gA, gB, sA, sB):
    with_deg = dg_o is not None
    c = lax.axis_index("c")
    s = lax.axis_index("s")
    w = c * NS + s

    ebase = w * EPT
    pltpu.sync_copy(srcr.at[pl.ds(ebase, EPT)], src_v)
    pltpu.sync_copy(dstr.at[pl.ds(ebase, EPT)], dst_v)

    _zero_fill(zx, 160, DH // 16)
    if with_deg:
        _zero_fill(zrow, RPS, DE // 16)
        ov = jnp.ones((16,), _f32)

        @pl.loop(0, CW, unroll=8)
        def _(r):
            ones_v[r, :] = ov

    base = s * RPS
    for k in range(RPS // 160):
        pltpu.sync_copy(zx, sx_s.at[pl.ds(base + k * 160, 160)])
    if with_deg:
        pltpu.sync_copy(zrow, dg_s.at[pl.ds(base, RPS)])
    plsc.subcore_barrier()

    def start_g(j, xb, sem):
        pltpu.async_copy(xm.at[src_v.at[pl.ds(j * CW, CW)]], xb, sem)

    def wait_g(xb, sem):
        pltpu.make_async_copy(xm.at[src_v.at[pl.ds(0, CW)]], xb, sem).wait()

    def start_s(j, xb, sem):
        idx = dst_v.at[pl.ds(j * CW, CW)]
        pltpu.async_copy(xb, sx_s.at[idx], sem, add=True)
        if with_deg:
            pltpu.async_copy(ones_v, dg_s.at[idx], sem, add=True)

    def wait_s(xb, sem):
        pltpu.make_async_copy(xb, sx_s.at[dst_v.at[pl.ds(0, CW)]], sem).wait()
        if with_deg:
            pltpu.make_async_copy(
                ones_v, dg_s.at[dst_v.at[pl.ds(0, CW)]], sem).wait()

    start_g(0, gbA, gA)
    start_g(1, gbB, gB)

    @pl.loop(0, (CH - 3) // 2)
    def _(t):
        j0 = 2 * t
        wait_g(gbA, gA)
        start_s(j0, gbA, sA)
        wait_g(gbB, gB)
        start_s(j0 + 1, gbB, sB)
        wait_s(gbA, sA)
        start_g(j0 + 2, gbA, gA)
        wait_s(gbB, sB)
        start_g(j0 + 3, gbB, gB)

    wait_g(gbA, gA)
    start_s(CH - 3, gbA, sA)
    wait_s(gbA, sA)
    start_g(CH - 1, gbA, gA)
    wait_g(gbB, gB)
    start_s(CH - 2, gbB, sB)
    wait_g(gbA, gA)
    start_s(CH - 1, gbA, sA)
    wait_s(gbA, sA)
    wait_s(gbB, sB)

    plsc.subcore_barrier()

    pltpu.sync_copy(sx_s.at[pl.ds(base, RPS)], sx_o.at[c, pl.ds(base, RPS)])
    if with_deg:
        pltpu.sync_copy(dg_s.at[pl.ds(base, RPS)], dg_o.at[c, pl.ds(base, RPS)])


def _seg_body_xd(xm, srcr, dstr, sx_o, dg_o,
                 src_v, dst_v, gbA, gbB, ones_v, zx, zrow,
                 sx_s, dg_s, gA, gB, sA, sB):
    _seg_gather_body(xm, srcr, dstr, sx_o, dg_o,
                     src_v, dst_v, gbA, gbB, ones_v, zx, zrow,
                     sx_s, dg_s, gA, gB, sA, sB)


def _seg_body_h(hm, srcr, dstr, sh_o,
                src_v, dst_v, gbA, gbB, zx, sh_s, gA, gB, sA, sB):
    _seg_gather_body(hm, srcr, dstr, sh_o, None,
                     src_v, dst_v, gbA, gbB, None, zx, None,
                     sh_s, None, gA, gB, sA, sB)


def _seg_body_se(dstr, ea_hbm, se_o,
                 dst_v, eaA, eaB, zrow, se_s, gA, gB, sA, sB):
    c = lax.axis_index("c")
    s = lax.axis_index("s")
    w = c * NS + s
    ebase = w * EPT
    pltpu.sync_copy(dstr.at[pl.ds(ebase, EPT)], dst_v)

    _zero_fill(zrow, RPS, DE // 16)
    base = s * RPS
    pltpu.sync_copy(zrow, se_s.at[pl.ds(base, RPS)])
    plsc.subcore_barrier()

    def start_g(j, eb, sem):
        off = pl.multiple_of(ebase + j * CW, 16)
        pltpu.async_copy(ea_hbm.at[pl.ds(off, CW)], eb, sem)

    def wait_g(eb, sem):
        pltpu.make_async_copy(ea_hbm.at[pl.ds(0, CW)], eb, sem).wait()

    def start_s(j, eb, sem):
        pltpu.async_copy(eb, se_s.at[dst_v.at[pl.ds(j * CW, CW)]], sem,
                         add=True)

    def wait_s(eb, sem):
        pltpu.make_async_copy(eb, se_s.at[dst_v.at[pl.ds(0, CW)]], sem).wait()

    start_g(0, eaA, gA)
    start_g(1, eaB, gB)

    @pl.loop(0, (CH - 3) // 2)
    def _(t):
        j0 = 2 * t
        wait_g(eaA, gA)
        start_s(j0, eaA, sA)
        wait_g(eaB, gB)
        start_s(j0 + 1, eaB, sB)
        wait_s(eaA, sA)
        start_g(j0 + 2, eaA, gA)
        wait_s(eaB, sB)
        start_g(j0 + 3, eaB, gB)

    wait_g(eaA, gA)
    start_s(CH - 3, eaA, sA)
    wait_s(eaA, sA)
    start_g(CH - 1, eaA, gA)
    wait_g(eaB, gB)
    start_s(CH - 2, eaB, sB)
    wait_g(eaA, gA)
    start_s(CH - 1, eaA, sA)
    wait_s(eaA, sA)
    wait_s(eaB, sB)

    plsc.subcore_barrier()
    pltpu.sync_copy(se_s.at[pl.ds(base, RPS)], se_o.at[c, pl.ds(base, RPS)])


def _seg1x(xm, srcr, dstr):
    return pl.kernel(
        _seg_body_xd,
        out_type=[
            jax.ShapeDtypeStruct((NC, NPAD, DH), _f32),
            jax.ShapeDtypeStruct((NC, NPAD, DE), _f32),
        ],
        mesh=_mesh,
        scratch_types=[
            pltpu.VMEM((EPT,), jnp.int32),
            pltpu.VMEM((EPT,), jnp.int32),
            pltpu.VMEM((CW, DH), _f32),
            pltpu.VMEM((CW, DH), _f32),
            pltpu.VMEM((CW, DE), _f32),
            pltpu.VMEM((160, DH), _f32),
            pltpu.VMEM((RPS, DE), _f32),
            pltpu.VMEM_SHARED((NPAD, DH), _f32),
            pltpu.VMEM_SHARED((NPAD, DE), _f32),
            pltpu.SemaphoreType.DMA,
            pltpu.SemaphoreType.DMA,
            pltpu.SemaphoreType.DMA,
            pltpu.SemaphoreType.DMA,
        ],
        compiler_params=_sc_params,
    )(xm, srcr, dstr)


def _seg1e(dstr, ea):
    return pl.kernel(
        _seg_body_se,
        out_type=[jax.ShapeDtypeStruct((NC, NPAD, DE), _f32)],
        mesh=_mesh,
        scratch_types=[
            pltpu.VMEM((EPT,), jnp.int32),
            pltpu.VMEM((CW, DE), _f32),
            pltpu.VMEM((CW, DE), _f32),
            pltpu.VMEM((RPS, DE), _f32),
            pltpu.VMEM_SHARED((NPAD, DE), _f32),
            pltpu.SemaphoreType.DMA,
            pltpu.SemaphoreType.DMA,
            pltpu.SemaphoreType.DMA,
            pltpu.SemaphoreType.DMA,
        ],
        compiler_params=_sc_params,
    )(dstr, ea)


def _seg2(hm, srcr, dstr):
    return pl.kernel(
        _seg_body_h,
        out_type=[jax.ShapeDtypeStruct((NC, NPAD, DH), _f32)],
        mesh=_mesh,
        scratch_types=[
            pltpu.VMEM((EPT,), jnp.int32),
            pltpu.VMEM((EPT,), jnp.int32),
            pltpu.VMEM((CW, DH), _f32),
            pltpu.VMEM((CW, DH), _f32),
            pltpu.VMEM((160, DH), _f32),
            pltpu.VMEM_SHARED((NPAD, DH), _f32),
            pltpu.SemaphoreType.DMA,
            pltpu.SemaphoreType.DMA,
            pltpu.SemaphoreType.DMA,
            pltpu.SemaphoreType.DMA,
        ],
        compiler_params=_sc_params,
    )(hm, srcr, dstr)


_R = 2000
_G = N // _R


def _dot(a, b):
    return jnp.dot(a, b, preferred_element_type=_f32)


def _tc0_body(x_r, wm_r, xm_r):
    xm_r[...] = _dot(x_r[...], wm_r[...][:DX])


def _tc0(x, Wmsg):
    return pl.pallas_call(
        _tc0_body,
        grid=(_G,),
        in_specs=[
            pl.BlockSpec((_R, DX), lambda i: (i, 0)),
            pl.BlockSpec((DX + DE, DH), lambda i: (0, 0)),
        ],
        out_specs=pl.BlockSpec((_R, DH), lambda i: (i, 0)),
        out_shape=jax.ShapeDtypeStruct((N, DH), _f32),
    )(x, Wmsg)


def _tc1_body(x_r, sxm_r, se_r, dg_r, wm_r, bm_r, ws_r, bs_r, wm2_r,
              h1_r, hm_r):
    deg = dg_r[0, :, 0:1] + dg_r[1, :, 0:1]
    inv = 1.0 / jnp.maximum(deg, 1.0)
    se = se_r[0] + se_r[1]
    sxm = sxm_r[0] + sxm_r[1]
    agg = (sxm + _dot(se, wm_r[...][DX:]) + deg * bm_r[...]) * inv
    h = jnp.maximum(_dot(x_r[...], ws_r[...]) + bs_r[...] + agg, 0.0)
    h1_r[...] = h
    hm_r[...] = _dot(h, wm2_r[...][:DH])


def _tc1(x, sxm, se, dg, Wmsg, bmsg, Wself, bself, Wmsg2):
    return pl.pallas_call(
        _tc1_body,
        grid=(_G,),
        in_specs=[
            pl.BlockSpec((_R, DX), lambda i: (i, 0)),
            pl.BlockSpec((NC, _R, DH), lambda i: (0, i, 0)),
            pl.BlockSpec((NC, _R, DE), lambda i: (0, i, 0)),
            pl.BlockSpec((NC, _R, DE), lambda i: (0, i, 0)),
            pl.BlockSpec((DX + DE, DH), lambda i: (0, 0)),
            pl.BlockSpec((1, DH), lambda i: (0, 0)),
            pl.BlockSpec((DX, DH), lambda i: (0, 0)),
            pl.BlockSpec((1, DH), lambda i: (0, 0)),
            pl.BlockSpec((DH + DE, DH), lambda i: (0, 0)),
        ],
        out_specs=(
            pl.BlockSpec((_R, DH), lambda i: (i, 0)),
            pl.BlockSpec((_R, DH), lambda i: (i, 0)),
        ),
        out_shape=(
            jax.ShapeDtypeStruct((N, DH), _f32),
            jax.ShapeDtypeStruct((N, DH), _f32),
        ),
    )(x, sxm, se, dg, Wmsg, bmsg, Wself, bself, Wmsg2)


def _tc2_body(h1_r, shm_r, se_r, dg_r, wm_r, bm_r, ws_r, bs_r,
              mw1, mb1, mw2, mb2, sw1, sb1, sw2, sb2,
              tw1, tb1, tw2, tb2, sw_o, tr_o, acc):
    i = pl.program_id(0)

    @pl.when(i == 0)
    def _():
        acc[...] = jnp.zeros_like(acc)

    deg = dg_r[0, :, 0:1] + dg_r[1, :, 0:1]
    inv = 1.0 / jnp.maximum(deg, 1.0)
    se = se_r[0] + se_r[1]
    shm = shm_r[0] + shm_r[1]
    agg = (shm + _dot(se, wm_r[...][DH:]) + deg * bm_r[...]) * inv
    h2 = jnp.maximum(_dot(h1_r[...], ws_r[...]) + bs_r[...] + agg, 0.0)
    acc[...] += jnp.sum(h2, axis=0, keepdims=True)

    @pl.when(i == _G - 1)
    def _():
        g = acc[...] * (1.0 / N)
        z = jnp.maximum(_dot(g, mw1[...]) + mb1[...], 0.0)
        z = jnp.maximum(_dot(z, mw2[...]) + mb2[...], 0.0)
        zs = jnp.maximum(_dot(z, sw1[...]) + sb1[...], 0.0)
        sw_o[...] = jax.nn.sigmoid(_dot(zs, sw2[...]) + sb2[...])
        zt = jnp.maximum(_dot(z, tw1[...]) + tb1[...], 0.0)
        tr_o[...] = jax.nn.sigmoid(_dot(zt, tw2[...]) + tb2[...])


def _tc2(h1, shm, se, dg, Wmsg, bmsg, Wself, bself,
         mW1, mb1, mW2, mb2, sW1, sb1, sW2, sb2, tW1, tb1, tW2, tb2):
    full = lambda a, b: pl.BlockSpec((a, b), lambda i: (0, 0))
    return pl.pallas_call(
        _tc2_body,
        grid=(_G,),
        in_specs=[
            pl.BlockSpec((_R, DH), lambda i: (i, 0)),
            pl.BlockSpec((NC, _R, DH), lambda i: (0, i, 0)),
            pl.BlockSpec((NC, _R, DE), lambda i: (0, i, 0)),
            pl.BlockSpec((NC, _R, DE), lambda i: (0, i, 0)),
            full(DH + DE, DH), full(1, DH), full(DH, DH), full(1, DH),
            full(DH, 256), full(1, 256), full(256, 128), full(1, 128),
            full(128, 64), full(1, 64), full(64, SW), full(1, SW),
            full(128, 64), full(1, 64), full(64, TR), full(1, TR),
        ],
        out_specs=(
            pl.BlockSpec((1, SW), lambda i: (0, 0)),
            pl.BlockSpec((1, TR), lambda i: (0, 0)),
        ),
        out_shape=(
            jax.ShapeDtypeStruct((1, SW), _f32),
            jax.ShapeDtypeStruct((1, TR), _f32),
        ),
        scratch_shapes=[pltpu.VMEM((1, DH), _f32)],
    )(h1, shm, se, dg, Wmsg, bmsg, Wself, bself,
      mW1, mb1, mW2, mb2, sW1, sb1, sW2, sb2, tW1, tb1, tW2, tb2)


def kernel(x, edge_index, edge_attr, g1_Wmsg, g1_bmsg, g1_Wself, g1_bself,
           g2_Wmsg, g2_bmsg, g2_Wself, g2_bself, mlp_W1, mlp_b1, mlp_W2,
           mlp_b2, sw_W1, sw_b1, sw_W2, sw_b2, tr_W1, tr_b1, tr_W2, tr_b2):
    srcr = edge_index[0]
    dstr = edge_index[1]
    row = lambda b: b.reshape(1, -1)

    xm = _tc0(x, g1_Wmsg)
    sxm, dg = _seg1x(xm, srcr, dstr)
    (se,) = _seg1e(dstr, edge_attr)
    h1, hm = _tc1(x, sxm, se, dg, g1_Wmsg, row(g1_bmsg), g1_Wself,
                  row(g1_bself), g2_Wmsg)
    (shm,) = _seg2(hm, srcr, dstr)
    sw, tr = _tc2(h1, shm, se, dg, g2_Wmsg, row(g2_bmsg), g2_Wself,
                  row(g2_bself), mlp_W1, row(mlp_b1), mlp_W2, row(mlp_b2),
                  sw_W1, row(sw_b1), sw_W2, row(sw_b2),
                  tr_W1, row(tr_b1), tr_W2, row(tr_b2))
    return (sw.reshape(-1), tr.reshape(-1))
